# baseline (speedup 1.0000x reference)
import numpy as np
import jax
import jax.numpy as jnp
from jax.experimental import pallas as pl
from jax.experimental.pallas import tpu as pltpu

_SIZES = (76, 38, 19)
_ANCHORS = {76: [[28, 28], [46, 45], [64, 66]],
            38: [[102, 74], [78, 115], [132, 113]],
            19: [[149, 163], [174, 268], [257, 176]]}
_POFF = {76: 0, 38: 76 * 76, 19: 76 * 76 + 38 * 38}
_NBOX = 3 * (76 * 76 + 38 * 38 + 19 * 19)


def _build_add(s: int) -> np.ndarray:
    n = s * s
    stride = float(608 // s)
    m = 1.05 * stride
    p = np.arange(n, dtype=np.float32)
    add = np.zeros((2, n), dtype=np.float32)
    add[0] = (np.mod(p, s) - 0.025) * stride + 0.5 * m
    add[1] = (np.floor_divide(p, s) - 0.025) * stride + 0.5 * m
    return add


_ADD = {s: _build_add(s) for s in _SIZES}


def _body(x76, x38, x19, p76, p38, p19, out_ref):
    for xr, pr, s in ((x76, p76, 76), (x38, p38, 38), (x19, p19, 19)):
        n = s * s
        m = 1.05 * float(608 // s)
        for a in range(3):
            t = xr[0, a]
            aw, ah = _ANCHORS[s][a]
            xy = jnp.tanh(t[0:2, :] * 0.5) * (0.5 * m) + pr[...]
            w = jnp.exp(t[2:3, :]) * float(aw)
            h = jnp.exp(t[3:4, :]) * float(ah)
            cl = jnp.tanh(t[4:85, :] * 0.5) * 0.5 + 0.5
            res = jnp.concatenate([xy, w, h, cl], axis=0).T
            out_ref[0:1, pl.Slice(3 * _POFF[s] + a, n, 3), :] = res[None]


def kernel(x0, x1, x2):
    b = x0.shape[0]
    xs = [x.reshape(b, 3, 85, s * s)
          for x, s in zip((x0, x1, x2), _SIZES)]
    adds = [jnp.asarray(_ADD[s]) for s in _SIZES]

    def xspec(s):
        return pl.BlockSpec((1, 3, 85, s * s), lambda i: (i, 0, 0, 0))

    def pspec(s):
        return pl.BlockSpec((2, s * s), lambda i: (0, 0))

    return pl.pallas_call(
        _body,
        grid=(b,),
        in_specs=[xspec(s) for s in _SIZES] + [pspec(s) for s in _SIZES],
        out_specs=pl.BlockSpec((1, _NBOX, 85), lambda i: (i, 0, 0)),
        out_shape=jax.ShapeDtypeStruct((b, _NBOX, 85), jnp.float32),
        compiler_params=pltpu.CompilerParams(
            dimension_semantics=("parallel",),
            allow_input_fusion=[True, True, True, False, False, False]),
    )(*xs, *adds)


# speedup vs baseline: 1.0011x; 1.0011x over previous
import numpy as np
import jax
import jax.numpy as jnp
from jax.experimental import pallas as pl
from jax.experimental.pallas import tpu as pltpu

_SIZES = (76, 38, 19)
_ANCHORS = {76: [[28, 28], [46, 45], [64, 66]],
            38: [[102, 74], [78, 115], [132, 113]],
            19: [[149, 163], [174, 268], [257, 176]]}
_POFF = {76: 0, 38: 76 * 76, 19: 76 * 76 + 38 * 38}
_NBOX = 3 * (76 * 76 + 38 * 38 + 19 * 19)


def _build_add(s: int) -> np.ndarray:
    n = s * s
    stride = float(608 // s)
    m = 1.05 * stride
    p = np.arange(n, dtype=np.float32)
    add = np.zeros((2, n), dtype=np.float32)
    add[0] = (np.mod(p, s) - 0.025) * stride + 0.5 * m
    add[1] = (np.floor_divide(p, s) - 0.025) * stride + 0.5 * m
    return add


_ADD = {s: _build_add(s) for s in _SIZES}


def _body(x76, x38, x19, p76, p38, p19, out_ref):
    for xr, pr, s in ((x76, p76, 76), (x38, p38, 38), (x19, p19, 19)):
        n = s * s
        m = 1.05 * float(608 // s)
        for a in range(3):
            t = xr[0, a]
            aw, ah = _ANCHORS[s][a]
            xy = jnp.tanh(t[0:2, :] * 0.5) * (0.5 * m) + pr[...]
            w = jnp.exp(t[2:3, :]) * float(aw)
            h = jnp.exp(t[3:4, :]) * float(ah)
            cl = jnp.tanh(t[4:85, :] * 0.5) * 0.5 + 0.5
            res = jnp.concatenate([xy, w, h, cl], axis=0).T
            out_ref[0:1, pl.Slice(3 * _POFF[s] + a, n, 3), :] = res[None]


def kernel(x0, x1, x2):
    b = x0.shape[0]
    xs = [x.reshape(b, 3, 85, s * s)
          for x, s in zip((x0, x1, x2), _SIZES)]
    adds = [jnp.asarray(_ADD[s]) for s in _SIZES]

    def xspec(s):
        return pl.BlockSpec((1, 3, 85, s * s), lambda i: (i, 0, 0, 0))

    def pspec(s):
        return pl.BlockSpec((2, s * s), lambda i: (0, 0))

    return pl.pallas_call(
        _body,
        grid=(b,),
        in_specs=[xspec(s) for s in _SIZES] + [pspec(s) for s in _SIZES],
        out_specs=pl.BlockSpec((1, _NBOX, 85), lambda i: (i, 0, 0)),
        out_shape=jax.ShapeDtypeStruct((b, _NBOX, 85), jnp.float32),
        compiler_params=pltpu.CompilerParams(
            dimension_semantics=("parallel",)),
    )(*xs, *adds)
